# baseline (speedup 1.0000x reference)
import functools
import jax
import jax.numpy as jnp
from jax import lax
from jax.experimental import pallas as pl
from jax.experimental.pallas import tpu as pltpu
from jax.experimental.pallas import tpu_sc as plsc

N = 10000
E = 320000
D = 128
NP = 10240
EP = 327680
NC = 2
NS = 16
NW = NC * NS
CHUNK = 128
NCHUNK = EP // (NW * CHUNK)
ACC_ROWS = 10112
ACC_PER_TILE = ACC_ROWS // NS

_sc_mesh = functools.partial(
    plsc.VectorSubcoreMesh, core_axis_name="c", subcore_axis_name="s")


def _deg_body(edges_hbm, deg_hbm, idx_v, ones_v, zeros_v, deg_sh, sem):
  cid = lax.axis_index("c")
  sid = lax.axis_index("s")
  wid = cid * NS + sid
  for g in range(CHUNK // 16):
    ones_v[pl.ds(g * 16, 16)] = jnp.ones((16,), jnp.float32)
  z16 = jnp.zeros((16,), jnp.float32)

  def zbody(k, _):
    zeros_v[pl.ds(k * 16, 16)] = z16
    return 0

  lax.fori_loop(0, (NP // NS) // 16, zbody, 0)
  pltpu.sync_copy(zeros_v, deg_sh.at[pl.ds(sid * (NP // NS), NP // NS)])
  pltpu.sync_copy(edges_hbm.at[pl.ds(wid * NCHUNK, NCHUNK)], idx_v)
  plsc.subcore_barrier()

  LAG = 4

  def body(j, _):
    pltpu.async_copy(ones_v, deg_sh.at[idx_v.at[j, 1]], sem, add=True)

    @pl.when(j >= LAG)
    def _():
      pltpu.make_async_copy(ones_v, deg_sh.at[idx_v.at[j - LAG, 1]],
                            sem).wait()

    return 0

  lax.fori_loop(0, NCHUNK, body, 0)

  def drain(j, _):
    pltpu.make_async_copy(ones_v, deg_sh.at[idx_v.at[j, 1]], sem).wait()
    return 0

  lax.fori_loop(NCHUNK - LAG, NCHUNK, drain, 0)
  plsc.subcore_barrier()

  @pl.when(sid == 0)
  def _():
    pltpu.sync_copy(deg_sh, deg_hbm.at[cid])


@jax.jit
def _deg_kernel(edges):
  return pl.kernel(
      _deg_body,
      out_type=jax.ShapeDtypeStruct((NC, NP), jnp.float32),
      mesh=_sc_mesh(),
      scratch_types=[
          pltpu.VMEM((NCHUNK, 2, CHUNK), jnp.int32),
          pltpu.VMEM((CHUNK,), jnp.float32),
          pltpu.VMEM((NP // NS,), jnp.float32),
          pltpu.VMEM_SHARED((NP,), jnp.float32),
          pltpu.SemaphoreType.DMA,
      ],
  )(edges)


def _matmul_body(f_ref, w_ref, degp_ref, y_ref):
  deg = degp_ref[0, :] + degp_ref[1, :] + 1.0
  dinv = lax.rsqrt(deg)
  x = jnp.dot(f_ref[...], w_ref[...], preferred_element_type=jnp.float32)
  y_ref[...] = x * dinv[:, None]


@jax.jit
def _matmul_kernel(features, W, deg_p):
  blk = 2048
  return pl.pallas_call(
      _matmul_body,
      grid=(NP // blk,),
      in_specs=[
          pl.BlockSpec((blk, D), lambda i: (i, 0)),
          pl.BlockSpec((D, D), lambda i: (0, 0)),
          pl.BlockSpec((NC, blk), lambda i: (0, i)),
      ],
      out_specs=pl.BlockSpec((blk, D), lambda i: (i, 0)),
      out_shape=jax.ShapeDtypeStruct((NP, D), jnp.float32),
  )(features, W, deg_p)


def _agg_body(y_hbm, edges_hbm, out_hbm,
              ir_v, ic_v, rows0_v, rows1_v, rows2_v, acc_sh, *sems):
  cid = lax.axis_index("c")
  sid = lax.axis_index("s")
  wid = cid * NS + sid
  g0 = wid * NCHUNK
  rows = (rows0_v, rows1_v, rows2_v)
  irsems = sems[0:3]
  icsems = sems[3:7]
  gsems = sems[7:10]
  ssems = sems[10:13]

  z16 = jnp.zeros((16,), jnp.float32)

  def zbody(k, _):
    for g in range(D // 16):
      rows0_v[k, pl.ds(g * 16, 16)] = z16
    return 0

  a0 = sid * ACC_PER_TILE

  @pl.when(cid == 0)
  def _():
    pltpu.sync_copy(y_hbm.at[pl.ds(a0, ACC_PER_TILE)],
                    acc_sh.at[pl.ds(a0, ACC_PER_TILE)])

  @pl.when(cid == 1)
  def _():
    lax.fori_loop(0, CHUNK, zbody, 0)
    off = 0
    while off < ACC_PER_TILE:
      step = min(CHUNK, ACC_PER_TILE - off)
      pltpu.sync_copy(rows0_v.at[pl.ds(0, step)],
                      acc_sh.at[pl.ds(a0 + off, step)])
      off += step

  def fetch_ir(j, s):
    pltpu.async_copy(edges_hbm.at[g0 + j, 0], ir_v.at[s], irsems[s])

  def wait_ir(j, s):
    pltpu.make_async_copy(edges_hbm.at[g0 + j, 0], ir_v.at[s],
                          irsems[s]).wait()

  def fetch_ic(j, c):
    pltpu.async_copy(edges_hbm.at[g0 + j, 1], ic_v.at[c], icsems[c])

  def wait_ic(j, c):
    pltpu.make_async_copy(edges_hbm.at[g0 + j, 1], ic_v.at[c],
                          icsems[c]).wait()

  def start_gather(s):
    pltpu.async_copy(y_hbm.at[ir_v.at[s]], rows[s], gsems[s])

  def wait_gather(s):
    pltpu.make_async_copy(y_hbm.at[ir_v.at[s]], rows[s], gsems[s]).wait()

  def start_scatter(s, c):
    pltpu.async_copy(rows[s], acc_sh.at[ic_v.at[c]], ssems[s], add=True)

  def wait_scatter(s, c):
    pltpu.make_async_copy(rows[s], acc_sh.at[ic_v.at[c]], ssems[s]).wait()

  def process(j, jd, skip_ws=False):
    s, s2, s3 = j % 3, (j + 2) % 3, (j + 3) % 3
    c, c2 = j % 4, (j + 2) % 4
    wait_ir(jd + 2, s2)
    if not skip_ws:
      wait_scatter(s2, (j - 1) % 4)
    start_gather(s2)
    wait_gather(s)
    if j + 3 < NCHUNK:
      fetch_ir(jd + 3, s3)
    wait_ic(jd, c)
    start_scatter(s, c)
    if j + 2 < NCHUNK:
      fetch_ic(jd + 2, c2)

  fetch_ir(0, 0)
  fetch_ic(0, 0)
  fetch_ir(1, 1)
  fetch_ic(1, 1)
  fetch_ir(2, 2)
  plsc.subcore_barrier()
  wait_ir(0, 0)
  start_gather(0)
  wait_ir(1, 1)
  start_gather(1)

  process(0, 0, skip_ws=True)
  UNROLL = 12
  STEADY = 72

  def body12(i, _):
    j0 = 1 + UNROLL * i
    for o in range(UNROLL):
      process(o + 1, j0 + o)
    return 0

  lax.fori_loop(0, STEADY // UNROLL, body12, 0)
  for j in range(1 + STEADY, NCHUNK - 2):
    process(j, j)
  for j in range(NCHUNK - 2, NCHUNK):
    wait_gather(j % 3)
    wait_ic(j, j % 4)
    start_scatter(j % 3, j % 4)
  for j in range(NCHUNK - 3, NCHUNK):
    wait_scatter(j % 3, j % 4)
  plsc.subcore_barrier()
  pltpu.sync_copy(acc_sh.at[pl.ds(a0, ACC_PER_TILE)],
                  out_hbm.at[cid, pl.ds(a0, ACC_PER_TILE)])


@jax.jit
def _agg_kernel(y, edges):
  return pl.kernel(
      _agg_body,
      out_type=jax.ShapeDtypeStruct((NC, ACC_ROWS, D), jnp.float32),
      mesh=_sc_mesh(),
      scratch_types=[
          pltpu.VMEM((3, CHUNK), jnp.int32),
          pltpu.VMEM((4, CHUNK), jnp.int32),
          pltpu.VMEM((CHUNK, D), jnp.float32),
          pltpu.VMEM((CHUNK, D), jnp.float32),
          pltpu.VMEM((CHUNK, D), jnp.float32),
          pltpu.VMEM_SHARED((ACC_ROWS, D), jnp.float32),
      ] + [pltpu.SemaphoreType.DMA] * 13,
  )(y, edges)


def _combine_body(p_ref, degp_ref, b_ref, o_ref):
  deg = degp_ref[0, :, 0] + degp_ref[1, :, 0] + 1.0
  dinv = lax.rsqrt(deg)
  s = p_ref[0] + p_ref[1]
  o_ref[...] = s * dinv[:, None] + b_ref[...]


@jax.jit
def _combine_kernel(partials, deg_p, b2d):
  blk = 2000
  return pl.pallas_call(
      _combine_body,
      grid=(N // blk,),
      in_specs=[
          pl.BlockSpec((NC, blk, D), lambda i: (0, i, 0)),
          pl.BlockSpec((NC, blk, 1), lambda i: (0, i, 0)),
          pl.BlockSpec((1, D), lambda i: (0, 0)),
      ],
      out_specs=pl.BlockSpec((blk, D), lambda i: (i, 0)),
      out_shape=jax.ShapeDtypeStruct((N, D), jnp.float32),
  )(partials, deg_p.reshape(NC, NP, 1), b2d)


def kernel(features, edge_index, W, b):
  ech = E // CHUNK
  npadc = (EP - E) // CHUNK
  inter = edge_index.reshape(2, ech, CHUNK).transpose(1, 0, 2)
  ar = jnp.arange(npadc * CHUNK, dtype=jnp.int32)
  pad_rows = (N + ar % (NP - N)).reshape(npadc, 1, CHUNK)
  pad_cols = (N + ar % (ACC_ROWS - N)).reshape(npadc, 1, CHUNK)
  edges = jnp.concatenate(
      [inter, jnp.concatenate([pad_rows, pad_cols], axis=1)], axis=0)

  deg_p = _deg_kernel(edges)
  y = _matmul_kernel(features, W, deg_p)
  partials = _agg_kernel(y, edges)
  return _combine_kernel(partials, deg_p, b.reshape(1, D))


# speedup vs baseline: 1.0358x; 1.0358x over previous
import functools
import jax
import jax.numpy as jnp
from jax import lax
from jax.experimental import pallas as pl
from jax.experimental.pallas import tpu as pltpu
from jax.experimental.pallas import tpu_sc as plsc

N = 10000
E = 320000
D = 128
NP = 10240
EP = 327680
NC = 2
NS = 16
NW = NC * NS
CHUNK = 128
NCHUNK = EP // (NW * CHUNK)
ACC_ROWS = 10112
ACC_PER_TILE = ACC_ROWS // NS

_sc_mesh = functools.partial(
    plsc.VectorSubcoreMesh, core_axis_name="c", subcore_axis_name="s")


ECHC = E // CHUNK
PADC = (EP - E) // CHUNK


def _deg_body(inter_hbm, pads_hbm, deg_hbm, idx_v, ones_v, zeros_v, deg_sh,
              sem):
  cid = lax.axis_index("c")
  sid = lax.axis_index("s")
  wid = cid * NS + sid
  for g in range(CHUNK // 16):
    ones_v[pl.ds(g * 16, 16)] = jnp.ones((16,), jnp.float32)
  z16 = jnp.zeros((16,), jnp.float32)

  def zbody(k, _):
    zeros_v[pl.ds(k * 16, 16)] = z16
    return 0

  lax.fori_loop(0, (NP // NS) // 16, zbody, 0)
  pltpu.sync_copy(zeros_v, deg_sh.at[pl.ds(sid * (NP // NS), NP // NS)])
  @pl.when(wid < NW - 1)
  def _():
    pltpu.sync_copy(inter_hbm.at[pl.ds(wid * NCHUNK, NCHUNK)], idx_v)

  @pl.when(wid == NW - 1)
  def _():
    nreal = ECHC - (NW - 1) * NCHUNK
    pltpu.sync_copy(inter_hbm.at[pl.ds(ECHC - nreal, nreal)],
                    idx_v.at[pl.ds(0, nreal)])
    pltpu.sync_copy(pads_hbm, idx_v.at[pl.ds(nreal, PADC)])

  plsc.subcore_barrier()

  LAG = 4

  def body(j, _):
    pltpu.async_copy(ones_v, deg_sh.at[idx_v.at[j, 1]], sem, add=True)

    @pl.when(j >= LAG)
    def _():
      pltpu.make_async_copy(ones_v, deg_sh.at[idx_v.at[j - LAG, 1]],
                            sem).wait()

    return 0

  lax.fori_loop(0, NCHUNK, body, 0)

  def drain(j, _):
    pltpu.make_async_copy(ones_v, deg_sh.at[idx_v.at[j, 1]], sem).wait()
    return 0

  lax.fori_loop(NCHUNK - LAG, NCHUNK, drain, 0)
  plsc.subcore_barrier()

  @pl.when(sid == 0)
  def _():
    pltpu.sync_copy(deg_sh, deg_hbm.at[cid])


@jax.jit
def _deg_kernel(inter, pads):
  return pl.kernel(
      _deg_body,
      out_type=jax.ShapeDtypeStruct((NC, NP), jnp.float32),
      mesh=_sc_mesh(),
      scratch_types=[
          pltpu.VMEM((NCHUNK, 2, CHUNK), jnp.int32),
          pltpu.VMEM((CHUNK,), jnp.float32),
          pltpu.VMEM((NP // NS,), jnp.float32),
          pltpu.VMEM_SHARED((NP,), jnp.float32),
          pltpu.SemaphoreType.DMA,
      ],
  )(inter, pads)


def _matmul_body(f_ref, w_ref, degp_ref, y_ref):
  deg = degp_ref[0, :] + degp_ref[1, :] + 1.0
  dinv = lax.rsqrt(deg)
  x = jnp.dot(f_ref[...], w_ref[...], preferred_element_type=jnp.float32)
  y_ref[...] = x * dinv[:, None]


@jax.jit
def _matmul_kernel(features, W, deg_p):
  blk = 2048
  return pl.pallas_call(
      _matmul_body,
      grid=(NP // blk,),
      in_specs=[
          pl.BlockSpec((blk, D), lambda i: (i, 0)),
          pl.BlockSpec((D, D), lambda i: (0, 0)),
          pl.BlockSpec((NC, blk), lambda i: (0, i)),
      ],
      out_specs=pl.BlockSpec((blk, D), lambda i: (i, 0)),
      out_shape=jax.ShapeDtypeStruct((NP, D), jnp.float32),
  )(features, W, deg_p)


def _agg_body(y_hbm, inter_hbm, pads_hbm, out_hbm,
              ir_v, ic_v, rows0_v, rows1_v, rows2_v, acc_sh, *sems):
  cid = lax.axis_index("c")
  sid = lax.axis_index("s")
  wid = cid * NS + sid
  g0 = wid * NCHUNK
  rows = (rows0_v, rows1_v, rows2_v)
  irsems = sems[0:3]
  icsems = sems[3:7]
  gsems = sems[7:10]
  ssems = sems[10:13]

  z16 = jnp.zeros((16,), jnp.float32)

  def zbody(k, _):
    for g in range(D // 16):
      rows0_v[k, pl.ds(g * 16, 16)] = z16
    return 0

  a0 = sid * ACC_PER_TILE

  @pl.when(cid == 0)
  def _():
    pltpu.sync_copy(y_hbm.at[pl.ds(a0, ACC_PER_TILE)],
                    acc_sh.at[pl.ds(a0, ACC_PER_TILE)])

  @pl.when(cid == 1)
  def _():
    lax.fori_loop(0, CHUNK, zbody, 0)
    off = 0
    while off < ACC_PER_TILE:
      step = min(CHUNK, ACC_PER_TILE - off)
      pltpu.sync_copy(rows0_v.at[pl.ds(0, step)],
                      acc_sh.at[pl.ds(a0 + off, step)])
      off += step

  def fetch_idx(j, r, dst, sem):
    gd = g0 + j

    @pl.when(gd < ECHC)
    def _():
      pltpu.async_copy(inter_hbm.at[gd, r], dst, sem)

    @pl.when(gd >= ECHC)
    def _():
      pltpu.async_copy(pads_hbm.at[gd - ECHC, r], dst, sem)

  def fetch_ir(j, s):
    fetch_idx(j, 0, ir_v.at[s], irsems[s])

  def wait_ir(j, s):
    del j
    pltpu.make_async_copy(inter_hbm.at[0, 0], ir_v.at[s], irsems[s]).wait()

  def fetch_ic(j, c):
    fetch_idx(j, 1, ic_v.at[c], icsems[c])

  def wait_ic(j, c):
    del j
    pltpu.make_async_copy(inter_hbm.at[0, 1], ic_v.at[c], icsems[c]).wait()

  def start_gather(s):
    pltpu.async_copy(y_hbm.at[ir_v.at[s]], rows[s], gsems[s])

  def wait_gather(s):
    pltpu.make_async_copy(y_hbm.at[ir_v.at[s]], rows[s], gsems[s]).wait()

  def start_scatter(s, c):
    pltpu.async_copy(rows[s], acc_sh.at[ic_v.at[c]], ssems[s], add=True)

  def wait_scatter(s, c):
    pltpu.make_async_copy(rows[s], acc_sh.at[ic_v.at[c]], ssems[s]).wait()

  def process(j, jd, skip_ws=False):
    s, s2, s3 = j % 3, (j + 2) % 3, (j + 3) % 3
    c, c2 = j % 4, (j + 2) % 4
    wait_ir(jd + 2, s2)
    if not skip_ws:
      wait_scatter(s2, (j - 1) % 4)
    start_gather(s2)
    wait_gather(s)
    if j + 3 < NCHUNK:
      fetch_ir(jd + 3, s3)
    wait_ic(jd, c)
    start_scatter(s, c)
    if j + 2 < NCHUNK:
      fetch_ic(jd + 2, c2)

  fetch_ir(0, 0)
  fetch_ic(0, 0)
  fetch_ir(1, 1)
  fetch_ic(1, 1)
  fetch_ir(2, 2)
  plsc.subcore_barrier()
  wait_ir(0, 0)
  start_gather(0)
  wait_ir(1, 1)
  start_gather(1)

  process(0, 0, skip_ws=True)
  UNROLL = 12
  STEADY = 72

  def body12(i, _):
    j0 = 1 + UNROLL * i
    for o in range(UNROLL):
      process(o + 1, j0 + o)
    return 0

  lax.fori_loop(0, STEADY // UNROLL, body12, 0)
  for j in range(1 + STEADY, NCHUNK - 2):
    process(j, j)
  for j in range(NCHUNK - 2, NCHUNK):
    wait_gather(j % 3)
    wait_ic(j, j % 4)
    start_scatter(j % 3, j % 4)
  for j in range(NCHUNK - 3, NCHUNK):
    wait_scatter(j % 3, j % 4)
  plsc.subcore_barrier()
  pltpu.sync_copy(acc_sh.at[pl.ds(a0, ACC_PER_TILE)],
                  out_hbm.at[cid, pl.ds(a0, ACC_PER_TILE)])


@jax.jit
def _agg_kernel(y, inter, pads):
  return pl.kernel(
      _agg_body,
      out_type=jax.ShapeDtypeStruct((NC, ACC_ROWS, D), jnp.float32),
      mesh=_sc_mesh(),
      scratch_types=[
          pltpu.VMEM((3, CHUNK), jnp.int32),
          pltpu.VMEM((4, CHUNK), jnp.int32),
          pltpu.VMEM((CHUNK, D), jnp.float32),
          pltpu.VMEM((CHUNK, D), jnp.float32),
          pltpu.VMEM((CHUNK, D), jnp.float32),
          pltpu.VMEM_SHARED((ACC_ROWS, D), jnp.float32),
      ] + [pltpu.SemaphoreType.DMA] * 13,
  )(y, inter, pads)


def _combine_body(p_ref, degp_ref, b_ref, o_ref):
  deg = degp_ref[0, :, 0] + degp_ref[1, :, 0] + 1.0
  dinv = lax.rsqrt(deg)
  s = p_ref[0] + p_ref[1]
  o_ref[...] = s * dinv[:, None] + b_ref[...]


@jax.jit
def _combine_kernel(partials, deg_p, b2d):
  blk = 2000
  return pl.pallas_call(
      _combine_body,
      grid=(N // blk,),
      in_specs=[
          pl.BlockSpec((NC, blk, D), lambda i: (0, i, 0)),
          pl.BlockSpec((NC, blk, 1), lambda i: (0, i, 0)),
          pl.BlockSpec((1, D), lambda i: (0, 0)),
      ],
      out_specs=pl.BlockSpec((blk, D), lambda i: (i, 0)),
      out_shape=jax.ShapeDtypeStruct((N, D), jnp.float32),
  )(partials, deg_p.reshape(NC, NP, 1), b2d)


def kernel(features, edge_index, W, b):
  inter = edge_index.reshape(2, ECHC, CHUNK).transpose(1, 0, 2)
  ar = jnp.arange(PADC * CHUNK, dtype=jnp.int32)
  pad_rows = (N + ar % (NP - N)).reshape(PADC, 1, CHUNK)
  pad_cols = (N + ar % (ACC_ROWS - N)).reshape(PADC, 1, CHUNK)
  pads = jnp.concatenate([pad_rows, pad_cols], axis=1)

  deg_p = _deg_kernel(inter, pads)
  y = _matmul_kernel(features, W, deg_p)
  partials = _agg_kernel(y, inter, pads)
  return _combine_kernel(partials, deg_p, b.reshape(1, D))
